# Initial kernel scaffold; baseline (speedup 1.0000x reference)
import jax
import jax.numpy as jnp
from jax.experimental import pallas as pl


def kernel(point_cloud, detect_point, feature4, feature5, feature6, fps_idx1, fps_idx2, W_ind_c, b_ind_c, W1_c, b1_c, W2_c, b2_c, W_ind_f, b_ind_f, W1_f, b1_f, W2_f, b2_f):
    raise NotImplementedError("write your pallas kernel here")


# speedup vs baseline: 6.8244x; 6.8244x over previous
import functools

import jax
import jax.numpy as jnp
from jax.experimental import pallas as pl
from jax.experimental.pallas import tpu as pltpu


_N_SUB = 8


def _main_kernel(n_np, k_close, k_far,
                 det_ref, pct_ref, f5_ref,
                 Wc_ref, bc_ref, W1c_ref, b1c_ref, W2c_ref, b2c_ref,
                 Wf_ref, bf_ref, W1f_ref, b1f_ref, W2f_ref, b2f_ref,
                 dis_ref, outc_ref, outf_ref,
                 t12_s, t24_s, dis_s):
    rb = det_ref.shape[1]
    px = pct_ref[0, 0:1, :]
    py = pct_ref[0, 1:2, :]
    pz = pct_ref[0, 2:3, :]

    def subtile(i, carry):
        qx = det_ref[0, pl.ds(i * _N_SUB, _N_SUB), 0:1]
        qy = det_ref[0, pl.ds(i * _N_SUB, _N_SUB), 1:2]
        qz = det_ref[0, pl.ds(i * _N_SUB, _N_SUB), 2:3]
        dx = qx - px
        dy = qy - py
        dz = qz - pz
        d2 = dx * dx + dy * dy + dz * dz
        dmin = jnp.min(d2, axis=1, keepdims=True)
        dis_s[pl.ds(i * _N_SUB, _N_SUB), :] = jnp.sqrt(dmin)

        def extract(t, c):
            work, t12, t24 = c
            m = jnp.min(work, axis=1, keepdims=True)
            t12 = jnp.where(t == k_far - 1, m, t12)
            t24 = jnp.where(t == k_close - 1, m, t24)
            work = jnp.where(work <= m, jnp.inf, work)
            return work, t12, t24

        init = jnp.zeros((_N_SUB, 1), jnp.float32)
        _, t12, t24 = jax.lax.fori_loop(0, k_close, extract, (d2, init, init))
        t12_s[pl.ds(i * _N_SUB, _N_SUB), :] = t12
        t24_s[pl.ds(i * _N_SUB, _N_SUB), :] = t24
        return carry

    jax.lax.fori_loop(0, rb // _N_SUB, subtile, 0)

    dis_ref[0] = dis_s[...]

    qx = det_ref[0, :, 0:1]
    qy = det_ref[0, :, 1:2]
    qz = det_ref[0, :, 2:3]
    dx = qx - px
    dy = qy - py
    dz = qz - pz
    d2 = dx * dx + dy * dy + dz * dz

    wfull = 1.0 / (jnp.sqrt(jnp.maximum(d2, 1e-10)) + 1e-8)
    w24 = jnp.where(d2 <= t24_s[...], wfull, 0.0)
    w24 = w24 / jnp.sum(w24, axis=1, keepdims=True)
    w12 = jnp.where(d2 <= t12_s[...], wfull, 0.0)
    w12 = w12 / jnp.sum(w12, axis=1, keepdims=True)

    f5 = f5_ref[0]
    hi = jax.lax.Precision.HIGHEST
    interp_c = jax.lax.dot(w24, f5, precision=hi)
    interp_f = jax.lax.dot(w12, f5, precision=hi)

    hc = jnp.maximum(jax.lax.dot(interp_c, Wc_ref[...], precision=hi) + bc_ref[...], 0.0)
    hc = jnp.maximum(jax.lax.dot(hc, W1c_ref[...], precision=hi) + b1c_ref[...], 0.0)
    lc = jnp.tanh(jax.lax.dot(hc, W2c_ref[...], precision=hi) + b2c_ref[...])
    outc_ref[0] = lc[:, 0:2]

    hf = jnp.maximum(jax.lax.dot(interp_f, Wf_ref[...], precision=hi) + bf_ref[...], 0.0)
    hf = jnp.maximum(jax.lax.dot(hf, W1f_ref[...], precision=hi) + b1f_ref[...], 0.0)
    lf = jnp.tanh(jax.lax.dot(hf, W2f_ref[...], precision=hi) + b2f_ref[...])
    outf_ref[0] = lf[:, 0:2]


def _select_kernel(n_total, n_close, rb2,
                   discol_ref, disrow_ref, outc_ref, outf_ref, out_ref):
    r = pl.program_id(1)
    dcol = discol_ref[0]
    drow = disrow_ref[0]
    icol = jax.lax.broadcasted_iota(jnp.int32, (rb2, 1), 0) + r * rb2
    irow = jax.lax.broadcasted_iota(jnp.int32, (1, n_total), 1)
    lt = (drow < dcol) | ((drow == dcol) & (irow < icol))
    rank = jnp.sum(lt.astype(jnp.int32), axis=1, keepdims=True)
    is_close = rank < n_close
    out_ref[0] = jnp.where(is_close, outc_ref[0], outf_ref[0])


def kernel(point_cloud, detect_point, feature4, feature5, feature6,
           fps_idx1, fps_idx2,
           W_ind_c, b_ind_c, W1_c, b1_c, W2_c, b2_c,
           W_ind_f, b_ind_f, W1_f, b1_f, W2_f, b2_f):
    B, N, _ = detect_point.shape
    Np = point_cloud.shape[1]
    C = feature5.shape[2]
    n_close = N * 2 // 3
    k_close, k_far = 24, 12
    RB = 256
    RB2 = 512

    pct = jnp.transpose(point_cloud, (0, 2, 1))
    b2 = lambda v: v.reshape(1, -1)

    grid = (B, N // RB)
    full = lambda s: pl.BlockSpec(s, lambda b, r: (0, 0))
    dis, out_c, out_f = pl.pallas_call(
        functools.partial(_main_kernel, Np, k_close, k_far),
        grid=grid,
        in_specs=[
            pl.BlockSpec((1, RB, 3), lambda b, r: (b, r, 0)),
            pl.BlockSpec((1, 3, Np), lambda b, r: (b, 0, 0)),
            pl.BlockSpec((1, Np, C), lambda b, r: (b, 0, 0)),
            full(W_ind_c.shape), full((1, b_ind_c.shape[0])),
            full(W1_c.shape), full((1, b1_c.shape[0])),
            full(W2_c.shape), full((1, b2_c.shape[0])),
            full(W_ind_f.shape), full((1, b_ind_f.shape[0])),
            full(W1_f.shape), full((1, b1_f.shape[0])),
            full(W2_f.shape), full((1, b2_f.shape[0])),
        ],
        out_specs=[
            pl.BlockSpec((1, RB, 1), lambda b, r: (b, r, 0)),
            pl.BlockSpec((1, RB, 2), lambda b, r: (b, r, 0)),
            pl.BlockSpec((1, RB, 2), lambda b, r: (b, r, 0)),
        ],
        out_shape=[
            jax.ShapeDtypeStruct((B, N, 1), jnp.float32),
            jax.ShapeDtypeStruct((B, N, 2), jnp.float32),
            jax.ShapeDtypeStruct((B, N, 2), jnp.float32),
        ],
        scratch_shapes=[
            pltpu.VMEM((RB, 1), jnp.float32),
            pltpu.VMEM((RB, 1), jnp.float32),
            pltpu.VMEM((RB, 1), jnp.float32),
        ],
    )(detect_point, pct, feature5,
      W_ind_c, b2(b_ind_c), W1_c, b2(b1_c), W2_c, b2(b2_c),
      W_ind_f, b2(b_ind_f), W1_f, b2(b1_f), W2_f, b2(b2_f))

    dis_col = dis
    dis_row = dis.reshape(B, 1, N)

    logit = pl.pallas_call(
        functools.partial(_select_kernel, N, n_close, RB2),
        grid=(B, N // RB2),
        in_specs=[
            pl.BlockSpec((1, RB2, 1), lambda b, r: (b, r, 0)),
            pl.BlockSpec((1, 1, N), lambda b, r: (b, 0, 0)),
            pl.BlockSpec((1, RB2, 2), lambda b, r: (b, r, 0)),
            pl.BlockSpec((1, RB2, 2), lambda b, r: (b, r, 0)),
        ],
        out_specs=pl.BlockSpec((1, RB2, 2), lambda b, r: (b, r, 0)),
        out_shape=jax.ShapeDtypeStruct((B, N, 2), jnp.float32),
    )(dis_col, dis_row, out_c, out_f)
    return logit


# speedup vs baseline: 15.9450x; 2.3365x over previous
import functools

import jax
import jax.numpy as jnp
from jax.experimental import pallas as pl
from jax.experimental.pallas import tpu as pltpu


_N_SUB = 8


def _main_kernel(n_np, k_close, k_far, n_bs_iters,
                 det_ref, pct_ref, f5_ref,
                 Wc_ref, bc_ref, W1c_ref, b1c_ref, W2c_ref, b2c_ref,
                 Wf_ref, bf_ref, W1f_ref, b1f_ref, W2f_ref, b2f_ref,
                 dis_ref, outc_ref, outf_ref,
                 d2i_s):
    rb = det_ref.shape[1]
    px = pct_ref[0, 0:1, :]
    py = pct_ref[0, 1:2, :]
    pz = pct_ref[0, 2:3, :]
    qx = det_ref[0, :, 0:1]
    qy = det_ref[0, :, 1:2]
    qz = det_ref[0, :, 2:3]
    dx = qx - px
    dy = qy - py
    dz = qz - pz
    d2 = dx * dx + dy * dy + dz * dz
    d2i = jax.lax.bitcast_convert_type(d2, jnp.int32)
    d2i_s[...] = d2i
    dminb = jnp.min(d2i, axis=1, keepdims=True)
    dis_ref[0] = jnp.sqrt(jax.lax.bitcast_convert_type(dminb, jnp.float32))

    inf_col = jnp.full((rb, 1), 0x7F800000, jnp.int32)

    def bs(t, c):
        lo24, hi24, lo12, hi12 = c
        m24 = lo24 + ((hi24 - lo24) >> 1)
        m12 = lo12 + ((hi12 - lo12) >> 1)
        d = d2i_s[...]
        c24 = jnp.sum((d <= m24).astype(jnp.int32), axis=1, keepdims=True)
        c12 = jnp.sum((d <= m12).astype(jnp.int32), axis=1, keepdims=True)
        ge24 = c24 >= k_close
        ge12 = c12 >= k_far
        return (jnp.where(ge24, lo24, m24 + 1), jnp.where(ge24, m24, hi24),
                jnp.where(ge12, lo12, m12 + 1), jnp.where(ge12, m12, hi12))

    _, t24, _, t12 = jax.lax.fori_loop(
        0, n_bs_iters, bs, (dminb, inf_col, dminb, inf_col))

    wfull = 1.0 / (jnp.sqrt(jnp.maximum(d2, 1e-10)) + 1e-8)
    w24 = jnp.where(d2i <= t24, wfull, 0.0)
    w24 = w24 / jnp.sum(w24, axis=1, keepdims=True)
    w12 = jnp.where(d2i <= t12, wfull, 0.0)
    w12 = w12 / jnp.sum(w12, axis=1, keepdims=True)

    f5 = f5_ref[0]
    hi = jax.lax.Precision.HIGHEST
    interp_c = jax.lax.dot(w24, f5, precision=hi)
    interp_f = jax.lax.dot(w12, f5, precision=hi)

    hc = jnp.maximum(jax.lax.dot(interp_c, Wc_ref[...], precision=hi) + bc_ref[...], 0.0)
    hc = jnp.maximum(jax.lax.dot(hc, W1c_ref[...], precision=hi) + b1c_ref[...], 0.0)
    lc = jnp.tanh(jax.lax.dot(hc, W2c_ref[...], precision=hi) + b2c_ref[...])
    outc_ref[0] = lc[:, 0:2]

    hf = jnp.maximum(jax.lax.dot(interp_f, Wf_ref[...], precision=hi) + bf_ref[...], 0.0)
    hf = jnp.maximum(jax.lax.dot(hf, W1f_ref[...], precision=hi) + b1f_ref[...], 0.0)
    lf = jnp.tanh(jax.lax.dot(hf, W2f_ref[...], precision=hi) + b2f_ref[...])
    outf_ref[0] = lf[:, 0:2]


def _select_kernel(n_total, n_close, rb2,
                   discol_ref, disrow_ref, outc_ref, outf_ref, out_ref):
    r = pl.program_id(1)
    dcol = discol_ref[0]
    drow = disrow_ref[0]
    icol = jax.lax.broadcasted_iota(jnp.int32, (rb2, 1), 0) + r * rb2
    irow = jax.lax.broadcasted_iota(jnp.int32, (1, n_total), 1)
    lt = (drow < dcol) | ((drow == dcol) & (irow < icol))
    rank = jnp.sum(lt.astype(jnp.int32), axis=1, keepdims=True)
    is_close = rank < n_close
    out_ref[0] = jnp.where(is_close, outc_ref[0], outf_ref[0])


def kernel(point_cloud, detect_point, feature4, feature5, feature6,
           fps_idx1, fps_idx2,
           W_ind_c, b_ind_c, W1_c, b1_c, W2_c, b2_c,
           W_ind_f, b_ind_f, W1_f, b1_f, W2_f, b2_f):
    B, N, _ = detect_point.shape
    Np = point_cloud.shape[1]
    C = feature5.shape[2]
    n_close = N * 2 // 3
    k_close, k_far = 24, 12
    RB = 256
    RB2 = 512

    pct = jnp.transpose(point_cloud, (0, 2, 1))
    b2 = lambda v: v.reshape(1, -1)

    grid = (B, N // RB)
    full = lambda s: pl.BlockSpec(s, lambda b, r: (0, 0))
    dis, out_c, out_f = pl.pallas_call(
        functools.partial(_main_kernel, Np, k_close, k_far, 22),
        grid=grid,
        in_specs=[
            pl.BlockSpec((1, RB, 3), lambda b, r: (b, r, 0)),
            pl.BlockSpec((1, 3, Np), lambda b, r: (b, 0, 0)),
            pl.BlockSpec((1, Np, C), lambda b, r: (b, 0, 0)),
            full(W_ind_c.shape), full((1, b_ind_c.shape[0])),
            full(W1_c.shape), full((1, b1_c.shape[0])),
            full(W2_c.shape), full((1, b2_c.shape[0])),
            full(W_ind_f.shape), full((1, b_ind_f.shape[0])),
            full(W1_f.shape), full((1, b1_f.shape[0])),
            full(W2_f.shape), full((1, b2_f.shape[0])),
        ],
        out_specs=[
            pl.BlockSpec((1, RB, 1), lambda b, r: (b, r, 0)),
            pl.BlockSpec((1, RB, 2), lambda b, r: (b, r, 0)),
            pl.BlockSpec((1, RB, 2), lambda b, r: (b, r, 0)),
        ],
        out_shape=[
            jax.ShapeDtypeStruct((B, N, 1), jnp.float32),
            jax.ShapeDtypeStruct((B, N, 2), jnp.float32),
            jax.ShapeDtypeStruct((B, N, 2), jnp.float32),
        ],
        scratch_shapes=[
            pltpu.VMEM((RB, Np), jnp.int32),
        ],
    )(detect_point, pct, feature5,
      W_ind_c, b2(b_ind_c), W1_c, b2(b1_c), W2_c, b2(b2_c),
      W_ind_f, b2(b_ind_f), W1_f, b2(b1_f), W2_f, b2(b2_f))

    dis_col = dis
    dis_row = dis.reshape(B, 1, N)

    logit = pl.pallas_call(
        functools.partial(_select_kernel, N, n_close, RB2),
        grid=(B, N // RB2),
        in_specs=[
            pl.BlockSpec((1, RB2, 1), lambda b, r: (b, r, 0)),
            pl.BlockSpec((1, 1, N), lambda b, r: (b, 0, 0)),
            pl.BlockSpec((1, RB2, 2), lambda b, r: (b, r, 0)),
            pl.BlockSpec((1, RB2, 2), lambda b, r: (b, r, 0)),
        ],
        out_specs=pl.BlockSpec((1, RB2, 2), lambda b, r: (b, r, 0)),
        out_shape=jax.ShapeDtypeStruct((B, N, 2), jnp.float32),
    )(dis_col, dis_row, out_c, out_f)
    return logit


# speedup vs baseline: 23.8578x; 1.4963x over previous
import functools

import jax
import jax.numpy as jnp
from jax.experimental import pallas as pl
from jax.experimental.pallas import tpu as pltpu


_N_SUB = 8


def _main_kernel(n_np, k_close, k_far, n_bs_iters,
                 det_ref, pct_ref, f5_ref,
                 Wc_ref, bc_ref, W1c_ref, b1c_ref, W2c_ref, b2c_ref,
                 Wf_ref, bf_ref, W1f_ref, b1f_ref, W2f_ref, b2f_ref,
                 dis_ref, outc_ref, outf_ref,
                 d2i_s):
    rb = det_ref.shape[1]
    px = pct_ref[0, 0:1, :]
    py = pct_ref[0, 1:2, :]
    pz = pct_ref[0, 2:3, :]
    qx = det_ref[0, :, 0:1]
    qy = det_ref[0, :, 1:2]
    qz = det_ref[0, :, 2:3]
    dx = qx - px
    dy = qy - py
    dz = qz - pz
    d2 = dx * dx + dy * dy + dz * dz
    d2i = jax.lax.bitcast_convert_type(d2, jnp.int32)
    d2i_s[...] = d2i
    dminb = jnp.min(d2i, axis=1, keepdims=True)
    dis_ref[0] = jnp.sqrt(jax.lax.bitcast_convert_type(dminb, jnp.float32))

    dmaxb = jnp.max(d2i, axis=1, keepdims=True)

    def bs(t, c):
        lo24, hi24, lo12, hi12 = c
        m24 = lo24 + ((hi24 - lo24) >> 1)
        m12 = lo12 + ((hi12 - lo12) >> 1)
        d = d2i_s[...]
        c24 = jnp.sum((d <= m24).astype(jnp.int32), axis=1, keepdims=True)
        c12 = jnp.sum((d <= m12).astype(jnp.int32), axis=1, keepdims=True)
        ge24 = c24 >= k_close
        ge12 = c12 >= k_far
        return (jnp.where(ge24, lo24, m24 + 1), jnp.where(ge24, m24, hi24),
                jnp.where(ge12, lo12, m12 + 1), jnp.where(ge12, m12, hi12))

    _, t24, _, t12 = jax.lax.fori_loop(
        0, n_bs_iters, bs, (dminb, dmaxb, dminb, dmaxb))

    wfull = 1.0 / (jnp.sqrt(jnp.maximum(d2, 1e-10)) + 1e-8)
    w24 = jnp.where(d2i <= t24, wfull, 0.0)
    w24 = w24 / jnp.sum(w24, axis=1, keepdims=True)
    w12 = jnp.where(d2i <= t12, wfull, 0.0)
    w12 = w12 / jnp.sum(w12, axis=1, keepdims=True)

    f5 = f5_ref[0]
    hi = jax.lax.Precision.DEFAULT
    interp_c = jax.lax.dot(w24, f5, precision=hi)
    interp_f = jax.lax.dot(w12, f5, precision=hi)

    hc = jnp.maximum(jax.lax.dot(interp_c, Wc_ref[...], precision=hi) + bc_ref[...], 0.0)
    hc = jnp.maximum(jax.lax.dot(hc, W1c_ref[...], precision=hi) + b1c_ref[...], 0.0)
    lc = jnp.tanh(jax.lax.dot(hc, W2c_ref[...], precision=hi) + b2c_ref[...])
    outc_ref[0] = lc[:, 0:2]

    hf = jnp.maximum(jax.lax.dot(interp_f, Wf_ref[...], precision=hi) + bf_ref[...], 0.0)
    hf = jnp.maximum(jax.lax.dot(hf, W1f_ref[...], precision=hi) + b1f_ref[...], 0.0)
    lf = jnp.tanh(jax.lax.dot(hf, W2f_ref[...], precision=hi) + b2f_ref[...])
    outf_ref[0] = lf[:, 0:2]


def _select_kernel(n_total, n_close, rb2,
                   discol_ref, disrow_ref, outc_ref, outf_ref, out_ref):
    r = pl.program_id(1)
    dcol = discol_ref[0]
    drow = disrow_ref[0]
    icol = jax.lax.broadcasted_iota(jnp.int32, (rb2, 1), 0) + r * rb2
    irow = jax.lax.broadcasted_iota(jnp.int32, (1, n_total), 1)
    lt = (drow < dcol) | ((drow == dcol) & (irow < icol))
    rank = jnp.sum(lt.astype(jnp.int32), axis=1, keepdims=True)
    is_close = rank < n_close
    out_ref[0] = jnp.where(is_close, outc_ref[0], outf_ref[0])


def kernel(point_cloud, detect_point, feature4, feature5, feature6,
           fps_idx1, fps_idx2,
           W_ind_c, b_ind_c, W1_c, b1_c, W2_c, b2_c,
           W_ind_f, b_ind_f, W1_f, b1_f, W2_f, b2_f):
    B, N, _ = detect_point.shape
    Np = point_cloud.shape[1]
    C = feature5.shape[2]
    n_close = N * 2 // 3
    k_close, k_far = 24, 12
    RB = 256
    RB2 = 512

    pct = jnp.transpose(point_cloud, (0, 2, 1))
    b2 = lambda v: v.reshape(1, -1)

    grid = (B, N // RB)
    full = lambda s: pl.BlockSpec(s, lambda b, r: (0, 0))
    dis, out_c, out_f = pl.pallas_call(
        functools.partial(_main_kernel, Np, k_close, k_far, 17),
        grid=grid,
        in_specs=[
            pl.BlockSpec((1, RB, 3), lambda b, r: (b, r, 0)),
            pl.BlockSpec((1, 3, Np), lambda b, r: (b, 0, 0)),
            pl.BlockSpec((1, Np, C), lambda b, r: (b, 0, 0)),
            full(W_ind_c.shape), full((1, b_ind_c.shape[0])),
            full(W1_c.shape), full((1, b1_c.shape[0])),
            full(W2_c.shape), full((1, b2_c.shape[0])),
            full(W_ind_f.shape), full((1, b_ind_f.shape[0])),
            full(W1_f.shape), full((1, b1_f.shape[0])),
            full(W2_f.shape), full((1, b2_f.shape[0])),
        ],
        out_specs=[
            pl.BlockSpec((1, RB, 1), lambda b, r: (b, r, 0)),
            pl.BlockSpec((1, RB, 2), lambda b, r: (b, r, 0)),
            pl.BlockSpec((1, RB, 2), lambda b, r: (b, r, 0)),
        ],
        out_shape=[
            jax.ShapeDtypeStruct((B, N, 1), jnp.float32),
            jax.ShapeDtypeStruct((B, N, 2), jnp.float32),
            jax.ShapeDtypeStruct((B, N, 2), jnp.float32),
        ],
        scratch_shapes=[
            pltpu.VMEM((RB, Np), jnp.int32),
        ],
    )(detect_point, pct, feature5,
      W_ind_c, b2(b_ind_c), W1_c, b2(b1_c), W2_c, b2(b2_c),
      W_ind_f, b2(b_ind_f), W1_f, b2(b1_f), W2_f, b2(b2_f))

    dis_col = dis
    dis_row = dis.reshape(B, 1, N)

    logit = pl.pallas_call(
        functools.partial(_select_kernel, N, n_close, RB2),
        grid=(B, N // RB2),
        in_specs=[
            pl.BlockSpec((1, RB2, 1), lambda b, r: (b, r, 0)),
            pl.BlockSpec((1, 1, N), lambda b, r: (b, 0, 0)),
            pl.BlockSpec((1, RB2, 2), lambda b, r: (b, r, 0)),
            pl.BlockSpec((1, RB2, 2), lambda b, r: (b, r, 0)),
        ],
        out_specs=pl.BlockSpec((1, RB2, 2), lambda b, r: (b, r, 0)),
        out_shape=jax.ShapeDtypeStruct((B, N, 2), jnp.float32),
    )(dis_col, dis_row, out_c, out_f)
    return logit
